# baseline (speedup 1.0000x reference)
import functools

import jax
import jax.numpy as jnp
from jax import lax
from jax.experimental import pallas as pl
from jax.experimental.pallas import tpu as pltpu
from jax.experimental.pallas import tpu_sc as plsc

N = 10000
DEG = 16
D = 128

NW = 32
NPAD = 10240
CHN = 8
NHALF = NPAD // 2
NCHA_TOT = (NHALF // 16) // CHN
NCHA_C0 = 24
NCHA_C1 = NCHA_TOT - NCHA_C0

SB = NPAD * DEG
PREAL = SB + NPAD
KC2 = 256
PPAD = 180224
PHALF = PPAD // 2
CHC_TOT = (PHALF // 16) // KC2
CHC_C0 = 13
CHC_C1 = CHC_TOT - CHC_C0

PADTS = 3.0e38
EPS = 1e-5

_f32 = jnp.float32
_i32 = jnp.int32


def _sc_mesh():
    return plsc.VectorSubcoreMesh(core_axis_name="c", subcore_axis_name="s")


_SC_PARAMS = pltpu.CompilerParams(needs_layout_passes=False)


def _sga_sorts(tid, sidx, uidx):
    for n in range(CHN):
        ts = plsc.bitcast(tid[n, pl.ds(0, DEG)], _f32)
        ids = tid[n, pl.ds(DEG, DEG)]
        _, si = plsc.sort_key_val(ts, ids)
        sidx[pl.ds(n * DEG, DEG)] = si
        uidx[pl.ds(n * DEG, DEG)] = ids


def _sort_gather_body(tid_hbm, x_hbm, nbtw_hbm, child_hbm, tsg_hbm,
                      tid0, tid1, tid2, tid3, sx0, sx1, ux0, ux1,
                      rx0, rx1, rt0, rt1,
                      st0, st1, st2, st3, sgx0, sgx1, sgt0, sgt1,
                      swx0, swx1, swt0, swt1):
    core = lax.axis_index("c")
    base = (lax.axis_index("s") * (NHALF // 16)
            + jnp.where(core == 1, NCHA_C0 * CHN, 0))
    nch = jnp.where(core == 1, NCHA_C1, NCHA_C0)
    tid = [tid0, tid1, tid2, tid3]
    st = [st0, st1, st2, st3]
    sidx = [sx0, sx1]
    uidx = [ux0, ux1]
    rx = [rx0, rx1]
    rt = [rt0, rt1]
    sgx = [sgx0, sgx1]
    sgt = [sgt0, sgt1]
    swx = [swx0, swx1]
    swt = [swt0, swt1]
    NR = CHN * DEG

    def tid_copy(c, s):
        pltpu.async_copy(tid_hbm.at[pl.ds(base + c * CHN, CHN)], tid[s], st[s])

    def tid_wait(s):
        pltpu.make_async_copy(tid_hbm.at[pl.ds(0, CHN)], tid[s], st[s]).wait()

    def gathers(b):
        pltpu.async_copy(x_hbm.at[sidx[b]], rx[b], sgx[b])
        pltpu.async_copy(nbtw_hbm.at[uidx[b]], rt[b], sgt[b])

    def gathers_wait(b):
        pltpu.make_async_copy(x_hbm.at[pl.ds(0, NR)], rx[b], sgx[b]).wait()
        pltpu.make_async_copy(nbtw_hbm.at[pl.ds(0, NR)], rt[b], sgt[b]).wait()

    def writes_wait(b):
        pltpu.make_async_copy(rx[b], child_hbm.at[pl.ds(0, NR)], swx[b]).wait()
        pltpu.make_async_copy(rt[b], tsg_hbm.at[pl.ds(0, NR)], swt[b]).wait()

    tid_copy(0, 0)
    tid_copy(1, 1)
    tid_wait(0)
    _sga_sorts(tid0, sx0, ux0)
    gathers(0)

    def outer(g, carry):
        for j in range(4):
            c = g * 4 + j
            s1, s2 = (j + 1) % 4, (j + 2) % 4
            b, b1 = j % 2, (j + 1) % 2

            @pl.when(c + 2 < nch)
            def _():
                tid_copy(c + 2, s2)

            @pl.when(c + 1 < nch)
            def _():
                tid_wait(s1)
                _sga_sorts(tid[s1], sidx[b1], uidx[b1])

                @pl.when(c >= 1)
                def _():
                    writes_wait(b1)

                gathers(b1)

            @pl.when(c < nch)
            def _():
                gathers_wait(b)
                nb = base + c * CHN
                pltpu.async_copy(
                    rx[b], child_hbm.at[pl.ds(nb * DEG, NR)], swx[b])
                pltpu.async_copy(
                    rt[b], tsg_hbm.at[pl.ds(nb * DEG, NR)], swt[b])
        return carry

    lax.fori_loop(0, (NCHA_C0 + 3) // 4, outer, 0)
    for b in range(2):
        writes_wait(b)


def _sort_gather(tid_half, xp, nbt_wide):
    f = pl.kernel(
        _sort_gather_body,
        out_type=(
            jax.ShapeDtypeStruct((NHALF * DEG, D), _f32),
            jax.ShapeDtypeStruct((NHALF * DEG, D), _f32),
        ),
        mesh=_sc_mesh(),
        scratch_types=(
            [pltpu.VMEM((CHN, 2 * DEG), _i32) for _ in range(4)]
            + [pltpu.VMEM((CHN * DEG,), _i32) for _ in range(4)]
            + [pltpu.VMEM((CHN * DEG, D), _f32) for _ in range(4)]
            + [pltpu.SemaphoreType.DMA for _ in range(12)]
        ),
        compiler_params=_SC_PARAMS,
    )
    return f(tid_half, xp, nbt_wide)


def _table_body(child_ref, x_ref, tsg_ref, nbt_ref, nts_ref, nbi_ref,
                table_ref, idxn_ref, idxs_ref, *, bn, noff):
    c3 = child_ref[...].reshape(bn, DEG, D)
    acc = x_ref[...]
    table_ref[0] = acc
    for k in range(1, DEG + 1):
        acc = acc + c3[:, k - 1, :]
        table_ref[k] = acc * (1.0 / (k + 1))
    tsg3 = tsg_ref[...].reshape(bn, DEG, D)[:, :, :DEG]
    t3 = nbt_ref[...][:, :, None]
    r = jnp.sum((tsg3 <= t3).astype(_i32), axis=2)
    idxn_ref[...] = r * NPAD + nbi_ref[...]
    rs = jnp.sum((nbt_ref[...] <= nts_ref[...]).astype(_i32), axis=1,
                 keepdims=True)
    nid = (lax.broadcasted_iota(_i32, (bn, 1), 0)
           + pl.program_id(0) * bn + noff)
    idxs_ref[...] = rs * NPAD + nid


def _build_table(child, tsg, xp, nbtp, ntp_col, nbip, half, prev=None):
    bn = 128
    grid = (NHALF // bn,)
    off = half * (NHALF // bn)

    def body(child_ref, x_ref, tsg_ref, nbt_ref, nts_ref, nbi_ref,
             *rest):
        _table_body(child_ref, x_ref, tsg_ref, nbt_ref, nts_ref, nbi_ref,
                    rest[-3], rest[-2], rest[-1], bn=bn, noff=half * NHALF)

    in_specs = [
        pl.BlockSpec((bn * DEG, D), lambda b: (b, 0)),
        pl.BlockSpec((bn, D), lambda b: (off + b, 0)),
        pl.BlockSpec((bn * DEG, D), lambda b: (b, 0)),
        pl.BlockSpec((bn, DEG), lambda b: (off + b, 0)),
        pl.BlockSpec((bn, 1), lambda b: (off + b, 0)),
        pl.BlockSpec((bn, DEG), lambda b: (off + b, 0)),
    ]
    args = [child, xp, tsg, nbtp, ntp_col, nbip]
    aliases = {}
    if prev is not None:
        in_specs += [pl.BlockSpec(memory_space=pl.ANY)] * 3
        args += list(prev)
        aliases = {6: 0, 7: 1, 8: 2}
    return pl.pallas_call(
        body,
        grid=grid,
        in_specs=in_specs,
        out_specs=[
            pl.BlockSpec((DEG + 1, bn, D), lambda b: (0, off + b, 0)),
            pl.BlockSpec((bn, DEG), lambda b: (off + b, 0)),
            pl.BlockSpec((bn, 1), lambda b: (off + b, 0)),
        ],
        out_shape=[
            jax.ShapeDtypeStruct((DEG + 1, NPAD, D), _f32),
            jax.ShapeDtypeStruct((NPAD, DEG), _i32),
            jax.ShapeDtypeStruct((NPAD, 1), _i32),
        ],
        input_output_aliases=aliases,
    )(*args)


def _tab_gather_body(idx_hbm, table_hbm, agg1_hbm,
                     ix00, ix01, ix10, ix11, outr0, outr1,
                     si0, si1, sg0, sg1, sw0, sw1):
    core = lax.axis_index("c")
    base = (lax.axis_index("s") * (PHALF // 16)
            + jnp.where(core == 1, CHC_C0 * KC2, 0))
    nch = jnp.where(core == 1, CHC_C1, CHC_C0)
    ix = [(ix00, ix01), (ix10, ix11)]
    si = [si0, si1]
    outr = [outr0, outr1]
    sg = [sg0, sg1]
    sw = [sw0, sw1]

    def idx_copy(c, b):
        pb = base + c * KC2
        pltpu.async_copy(idx_hbm.at[pl.ds(pb, 128)], ix[b][0], si[b])
        pltpu.async_copy(idx_hbm.at[pl.ds(pb + 128, 128)], ix[b][1], si[b])

    def idx_wait(b):
        for h in range(2):
            pltpu.make_async_copy(
                idx_hbm.at[pl.ds(0, 128)], ix[b][h], si[b]).wait()

    def tgather(b):
        for h in range(2):
            pltpu.async_copy(
                table_hbm.at[ix[b][h]], outr[b].at[pl.ds(h * 128, 128)],
                sg[b])

    def tgather_wait(b):
        for h in range(2):
            pltpu.make_async_copy(
                table_hbm.at[pl.ds(0, 128)], outr[b].at[pl.ds(h * 128, 128)],
                sg[b]).wait()

    idx_copy(0, 0)
    idx_copy(1, 1)
    idx_wait(0)
    tgather(0)

    def outer(g, carry):
        for j in range(2):
            c = g * 2 + j
            b, b1 = j, (j + 1) % 2

            @pl.when(c + 1 < nch)
            def _():
                idx_wait(b1)

                @pl.when(c >= 1)
                def _():
                    pltpu.make_async_copy(
                        outr[b1], agg1_hbm.at[pl.ds(0, KC2)], sw[b1]).wait()

                tgather(b1)

            @pl.when(c < nch)
            def _():
                tgather_wait(b)
                pltpu.async_copy(
                    outr[b], agg1_hbm.at[pl.ds(base + c * KC2, KC2)], sw[b])

            @pl.when(c + 2 < nch)
            def _():
                idx_copy(c + 2, b)
        return carry

    lax.fori_loop(0, (CHC_C0 + 1) // 2, outer, 0)
    for b in range(2):
        pltpu.make_async_copy(
            outr[b], agg1_hbm.at[pl.ds(0, KC2)], sw[b]).wait()


def _tab_gather(idx_half, table_flat):
    f = pl.kernel(
        _tab_gather_body,
        out_type=jax.ShapeDtypeStruct((PHALF, D), _f32),
        mesh=_sc_mesh(),
        scratch_types=(
            [pltpu.VMEM((128,), _i32) for _ in range(4)]
            + [pltpu.VMEM((KC2, D), _f32) for _ in range(2)]
            + [pltpu.SemaphoreType.DMA for _ in range(6)]
        ),
        compiler_params=_SC_PARAMS,
    )
    return f(idx_half, table_flat)


def _mm1_body(agg_ref, w1_ref, b1_ref, pw_ref, *rest):
    h_ref, st_ref, acc = rest[-3], rest[-2], rest[-1]
    step = pl.program_id(0)

    @pl.when(step == 0)
    def _():
        acc[...] = jnp.zeros_like(acc)

    a = agg_ref[...]
    h = jnp.dot(a, w1_ref[...], preferred_element_type=_f32) + b1_ref[...]
    h_ref[...] = h
    w = pw_ref[...]
    hw = h * w
    s1 = jnp.sum(hw, axis=0, keepdims=True)
    s2 = jnp.sum(h * hw, axis=0, keepdims=True)
    d = jnp.sum(w)
    acc[0:1] = acc[0:1] + s1
    acc[1:2] = acc[1:2] + s2
    acc[2:3] = acc[2:3] + jnp.full((1, D), d, _f32)
    st_ref[...] = acc[...]


def _mm1(agg1_h, W1, b1, pw_h, half, prev_h1=None):
    bp = 4096
    grid = (PHALF // bp,)
    hoff = half * (PHALF // bp)
    in_specs = [
        pl.BlockSpec((bp, D), lambda b: (b, 0)),
        pl.BlockSpec((D, D), lambda b: (0, 0)),
        pl.BlockSpec((1, D), lambda b: (0, 0)),
        pl.BlockSpec((bp, 1), lambda b: (b, 0)),
    ]
    args = [agg1_h, W1, b1, pw_h]
    aliases = {}
    if prev_h1 is not None:
        in_specs.append(pl.BlockSpec(memory_space=pl.ANY))
        args.append(prev_h1)
        aliases = {4: 0}
    return pl.pallas_call(
        _mm1_body,
        grid=grid,
        in_specs=in_specs,
        out_specs=[
            pl.BlockSpec((bp, D), lambda b: (hoff + b, 0)),
            pl.BlockSpec((8, D), lambda b: (0, 0)),
        ],
        out_shape=[
            jax.ShapeDtypeStruct((PPAD, D), _f32),
            jax.ShapeDtypeStruct((8, D), _f32),
        ],
        scratch_shapes=[pltpu.VMEM((8, D), _f32)],
        input_output_aliases=aliases,
    )(*args)


def _layer2_body(hn_ref, hs_ref, wn_ref, m2_ref, st1_ref, g1_ref, be1_ref,
                 w2_ref, b2_ref, h2_ref, st2_ref, acc, scr, *, bn):
    step = pl.program_id(0)

    @pl.when(step == 0)
    def _():
        acc[...] = jnp.zeros_like(acc)

    den = jnp.maximum(st1_ref[2:3], 1.0)
    mean = st1_ref[0:1] / den
    var = st1_ref[1:2] / den - mean * mean
    sc = g1_ref[...] * lax.rsqrt(var + EPS)
    sh = be1_ref[...] - mean * sc

    hw = jax.nn.relu(hn_ref[...] * sc + sh) * wn_ref[...]
    scr[...] = hw.reshape(bn, DEG, D)
    num = jax.nn.relu(hs_ref[...] * sc + sh)
    for k in range(DEG):
        num = num + scr[:, k, :]
    degs = jnp.sum(m2_ref[...], axis=1, keepdims=True) + 1.0
    agg2 = num / degs
    h2p = jnp.dot(agg2, w2_ref[...], preferred_element_type=_f32) + b2_ref[...]
    h2_ref[...] = h2p

    rid = lax.broadcasted_iota(_i32, (bn, 1), 0) + step * bn
    w = (rid < N).astype(_f32)
    hw = h2p * w
    acc[0:1] = acc[0:1] + jnp.sum(hw, axis=0, keepdims=True)
    acc[1:2] = acc[1:2] + jnp.sum(h2p * hw, axis=0, keepdims=True)
    acc[2:3] = acc[2:3] + jnp.full((1, D), jnp.sum(w), _f32)
    st2_ref[...] = acc[...]


def _layer2(h1pre, wnbr, mask2, stats1, gamma1, beta1, W2, b2):
    bn = 256
    grid = (NPAD // bn,)
    sb_blk = SB // bn
    return pl.pallas_call(
        functools.partial(_layer2_body, bn=bn),
        grid=grid,
        in_specs=[
            pl.BlockSpec((bn * DEG, D), lambda b: (b, 0)),
            pl.BlockSpec((bn, D), lambda b: (sb_blk + b, 0)),
            pl.BlockSpec((bn * DEG, 1), lambda b: (b, 0)),
            pl.BlockSpec((bn, DEG), lambda b: (b, 0)),
            pl.BlockSpec((8, D), lambda b: (0, 0)),
            pl.BlockSpec((1, D), lambda b: (0, 0)),
            pl.BlockSpec((1, D), lambda b: (0, 0)),
            pl.BlockSpec((D, D), lambda b: (0, 0)),
            pl.BlockSpec((1, D), lambda b: (0, 0)),
        ],
        out_specs=[
            pl.BlockSpec((bn, D), lambda b: (b, 0)),
            pl.BlockSpec((8, D), lambda b: (0, 0)),
        ],
        out_shape=[
            jax.ShapeDtypeStruct((NPAD, D), _f32),
            jax.ShapeDtypeStruct((8, D), _f32),
        ],
        scratch_shapes=[pltpu.VMEM((8, D), _f32),
                        pltpu.VMEM((bn, DEG, D), _f32)],
    )(h1pre, h1pre, wnbr, mask2, stats1, gamma1, beta1, W2, b2)


def _final_body(h2_ref, st2_ref, g2_ref, be2_ref, out_ref):
    den = jnp.maximum(st2_ref[2:3], 1.0)
    mean = st2_ref[0:1] / den
    var = st2_ref[1:2] / den - mean * mean
    sc = g2_ref[...] * lax.rsqrt(var + EPS)
    sh = be2_ref[...] - mean * sc
    out_ref[...] = jax.nn.relu(h2_ref[...] * sc + sh)


def _final(h2pre, stats2, gamma2, beta2):
    bn = 512
    grid = (NPAD // bn,)
    return pl.pallas_call(
        _final_body,
        grid=grid,
        in_specs=[
            pl.BlockSpec((bn, D), lambda b: (b, 0)),
            pl.BlockSpec((8, D), lambda b: (0, 0)),
            pl.BlockSpec((1, D), lambda b: (0, 0)),
            pl.BlockSpec((1, D), lambda b: (0, 0)),
        ],
        out_specs=pl.BlockSpec((bn, D), lambda b: (b, 0)),
        out_shape=jax.ShapeDtypeStruct((NPAD, D), _f32),
    )(h2pre, stats2, gamma2, beta2)


@jax.jit
def _run(x, node_ts, nbr_ts, W1, b1, gamma1, beta1, W2, b2, gamma2, beta2,
         nbr_idx):
    xp = jnp.zeros((NPAD, D), _f32).at[:N].set(x)
    ntp = jnp.zeros((NPAD,), _f32).at[:N].set(node_ts)
    nbtp = jnp.full((NPAD, DEG), PADTS, _f32).at[:N].set(nbr_ts)
    nbip = jnp.zeros((NPAD, DEG), _i32).at[:N].set(nbr_idx.astype(_i32))
    mask2 = (nbtp <= ntp[:, None]).astype(_f32)
    wself = jnp.zeros((NPAD,), _f32).at[:N].set(1.0)
    zpad_i = jnp.zeros((PPAD - PREAL,), _i32)
    zpad_f = jnp.zeros((PPAD - PREAL,), _f32)
    pair_w = jnp.concatenate([mask2.reshape(-1), wself, zpad_f])

    tid_packed = jnp.concatenate(
        [lax.bitcast_convert_type(nbtp, _i32), nbip], axis=1)
    nbt_wide = jnp.pad(nbtp, ((0, 0), (0, D - DEG)))

    ntp_col = ntp.reshape(NPAD, 1)
    child1, tsg1 = _sort_gather(tid_packed[:NHALF], xp, nbt_wide)
    child2, tsg2 = _sort_gather(tid_packed[NHALF:], xp, nbt_wide)
    prev = _build_table(child1, tsg1, xp, nbtp, ntp_col, nbip, 0)
    table, idxn, idxs = _build_table(child2, tsg2, xp, nbtp, ntp_col,
                                     nbip, 1, prev)
    table_flat = table.reshape((DEG + 1) * NPAD, D)
    idx_flat = jnp.concatenate(
        [idxn.reshape(-1), idxs.reshape(-1), zpad_i])
    pw_col = pair_w.reshape(PPAD, 1)
    agg1_a = _tab_gather(idx_flat[:PHALF], table_flat)
    agg1_b = _tab_gather(idx_flat[PHALF:], table_flat)
    h1_a, st_a = _mm1(agg1_a, W1, b1.reshape(1, D), pw_col[:PHALF], 0)
    h1pre, st_b = _mm1(agg1_b, W1, b1.reshape(1, D), pw_col[PHALF:], 1, h1_a)
    stats1 = st_a + st_b
    h2pre, stats2 = _layer2(h1pre, mask2.reshape(SB, 1), mask2, stats1,
                            gamma1.reshape(1, D), beta1.reshape(1, D),
                            W2, b2.reshape(1, D))
    h2 = _final(h2pre, stats2, gamma2.reshape(1, D), beta2.reshape(1, D))
    return h2[:N]


def kernel(x, node_ts, nbr_ts, W1, b1, gamma1, beta1, W2, b2, gamma2, beta2,
           nbr_idx):
    return _run(x, node_ts, nbr_ts, W1, b1, gamma1, beta1, W2, b2,
                gamma2, beta2, nbr_idx)
